# Initial kernel scaffold; baseline (speedup 1.0000x reference)
import jax
import jax.numpy as jnp
from jax.experimental import pallas as pl


def kernel(seq_sparse_idx, seq_dense, sent_sparse_idx, sent_dense, sequence_feature_lengths, W_seq, W_sent):
    raise NotImplementedError("write your pallas kernel here")


# speedup vs baseline: 1.0170x; 1.0170x over previous
import jax
import jax.numpy as jnp
from jax import lax
from jax.experimental import pallas as pl
from jax.experimental.pallas import tpu as pltpu
from jax.experimental.pallas import tpu_sc as plsc

B, T, V, D, DU = 1024, 50, 100000, 128, 256
U = D + DU
TP1 = T + 1
BLK = TP1 * U
NNZ_SEQ = 2
NNZ_SENT = 4
NIDX = T * NNZ_SEQ
NW = 32
BPW = B // NW
L = 16


def _sc_body(seq_idx, seq_dense, sent_idx, sent_dense, lens_b,
             wseq, wsent, comb, masko,
             idx_st, sidx_st, lens_bst, g, gs, d, ds, srow, o, mst, sem):
    cid = lax.axis_index("c")
    sid = lax.axis_index("s")
    wid = sid * 2 + cid
    b0 = wid * BPW

    pltpu.sync_copy(seq_idx.at[pl.ds(b0, BPW)], idx_st)
    pltpu.sync_copy(sent_idx.at[pl.ds(b0, BPW)], sidx_st)
    pltpu.sync_copy(lens_b.at[pl.ds(b0, BPW)], lens_bst)

    iota = jnp.arange(L, dtype=jnp.int32)
    zero_f = jnp.zeros((L,), jnp.float32)
    one_f = jnp.float32(1.0)
    zero_s = jnp.float32(0.0)

    def b_body(j, carry):
        b = b0 + j
        pltpu.async_copy(wseq.at[idx_st.at[j]], g, sem).wait()
        pltpu.async_copy(wsent.at[sidx_st.at[j]], gs, sem).wait()
        pltpu.sync_copy(seq_dense.at[b], d)
        pltpu.sync_copy(sent_dense.at[b], ds)
        len_v = lens_bst[j, pl.ds(0, L)]

        for cc in range(D // L):
            srow[pl.ds(cc * L, L)] = (
                gs[0, pl.ds(cc * L, L)] + gs[1, pl.ds(cc * L, L)]
                + gs[2, pl.ds(cc * L, L)] + gs[3, pl.ds(cc * L, L)])
        for cc in range(DU // L):
            srow[pl.ds(D + cc * L, L)] = ds[0, pl.ds(cc * L, L)]

        def t_body(t, c2):
            tv = jnp.full((L,), t, jnp.int32)
            m_lt = jnp.where(tv < len_v, one_f, zero_s)
            m_eq = jnp.where(tv == len_v, one_f, zero_s)
            for cc in range(D // L):
                v = (g[2 * t, pl.ds(cc * L, L)]
                     + g[2 * t + 1, pl.ds(cc * L, L)])
                o[pl.ds(t * U + cc * L, L)] = (
                    v * m_lt + srow[pl.ds(cc * L, L)] * m_eq)
            for cc in range(DU // L):
                o[pl.ds(t * U + D + cc * L, L)] = (
                    d[t, pl.ds(cc * L, L)] * m_lt
                    + srow[pl.ds(D + cc * L, L)] * m_eq)
            return c2

        lax.fori_loop(0, T, t_body, 0)

        for cc in range(U // L):
            o[pl.ds(T * U + cc * L, L)] = zero_f

        pltpu.sync_copy(o, comb.at[pl.ds(b * BLK, BLK)])

        lp1 = jnp.full((L,), 1, jnp.int32) + len_v
        for k in range(4):
            tvec = k * L + iota
            m = jnp.where(tvec < lp1, one_f, zero_s)
            mst[pl.ds(j * TP1 + k * L, L)] = m
        return carry

    lax.fori_loop(0, BPW, b_body, 0)

    pltpu.sync_copy(mst.at[pl.ds(0, BPW * TP1)],
                    masko.at[pl.ds(b0 * TP1, BPW * TP1)])


@jax.jit
def _run(seq_idx, seq_dense, sent_idx, sent_dense, lens_b, wseq, wsent):
    mesh = plsc.VectorSubcoreMesh(core_axis_name="c", subcore_axis_name="s")
    return pl.kernel(
        _sc_body,
        mesh=mesh,
        out_type=[
            jax.ShapeDtypeStruct((B * BLK,), jnp.float32),
            jax.ShapeDtypeStruct((B * TP1,), jnp.float32),
        ],
        scratch_types=[
            pltpu.VMEM((BPW, NIDX), jnp.int32),
            pltpu.VMEM((BPW, NNZ_SENT), jnp.int32),
            pltpu.VMEM((BPW, L), jnp.int32),
            pltpu.VMEM((NIDX, D), jnp.float32),
            pltpu.VMEM((NNZ_SENT, D), jnp.float32),
            pltpu.VMEM((T, DU), jnp.float32),
            pltpu.VMEM((1, DU), jnp.float32),
            pltpu.VMEM((U,), jnp.float32),
            pltpu.VMEM((BLK,), jnp.float32),
            pltpu.VMEM((BPW * TP1 + L,), jnp.float32),
            pltpu.SemaphoreType.DMA,
        ],
    )(seq_idx, seq_dense, sent_idx, sent_dense, lens_b, wseq, wsent)


def kernel(seq_sparse_idx, seq_dense, sent_sparse_idx, sent_dense,
           sequence_feature_lengths, W_seq, W_sent):
    seq_idx = seq_sparse_idx.reshape(B, NIDX).astype(jnp.int32)
    sent_idx = sent_sparse_idx.reshape(B, NNZ_SENT).astype(jnp.int32)
    lens = sequence_feature_lengths.astype(jnp.int32)
    lens_b = jnp.broadcast_to(lens[:, None], (B, L))
    comb_flat, mask_flat = _run(seq_idx, seq_dense, sent_idx, sent_dense,
                                lens_b, W_seq, W_sent)
    return comb_flat.reshape(B, TP1, U), mask_flat.reshape(B, TP1, 1)


# speedup vs baseline: 1.2787x; 1.2573x over previous
import jax
import jax.numpy as jnp
from jax import lax
from jax.experimental import pallas as pl
from jax.experimental.pallas import tpu as pltpu
from jax.experimental.pallas import tpu_sc as plsc

B, T, V, D, DU = 1024, 50, 100000, 128, 256
U = D + DU
TP1 = T + 1
BLK = TP1 * U
NNZ_SEQ = 2
NNZ_SENT = 4
NIDX = T * NNZ_SEQ
NW = 32
BPW = B // NW
L = 16
NBUF = 2


def _sc_body(seq_idx, seq_dense, sent_idx, sent_dense, lens_b,
             wseq, wsent, comb, masko,
             idx_st, sidx_st, lens_bst, gsall, dsall,
             g0, g1, d0, d1, o0, o1, srow, mst,
             sgs, sg0, sg1, sd0, sd1, so0, so1):
    cid = lax.axis_index("c")
    sid = lax.axis_index("s")
    wid = sid * 2 + cid
    b0 = wid * BPW

    gbuf = (g0, g1)
    dbuf = (d0, d1)
    obuf = (o0, o1)
    sg = (sg0, sg1)
    sd = (sd0, sd1)
    so = (so0, so1)

    pltpu.sync_copy(seq_idx.at[pl.ds(b0, BPW)], idx_st)
    pltpu.sync_copy(sent_idx.at[pl.ds(b0 * NNZ_SENT, BPW * NNZ_SENT)],
                    sidx_st)
    pltpu.sync_copy(lens_b.at[pl.ds(b0, BPW)], lens_bst)
    pltpu.sync_copy(sent_dense.at[pl.ds(b0, BPW)], dsall)
    pltpu.async_copy(wsent.at[sidx_st], gsall, sgs).wait()

    iota = jnp.arange(L, dtype=jnp.int32)
    zero_f = jnp.zeros((L,), jnp.float32)
    one_f = jnp.float32(1.0)
    zero_s = jnp.float32(0.0)

    for k in range(NBUF):
        pltpu.async_copy(wseq.at[idx_st.at[k]], gbuf[k], sg[k])
        pltpu.async_copy(seq_dense.at[b0 + k], dbuf[k], sd[k])

    def pair_body(p, carry):
        for k in range(NBUF):
            j = p * NBUF + k
            b = b0 + j
            gk, dk, ok = gbuf[k], dbuf[k], obuf[k]

            pltpu.make_async_copy(wseq.at[idx_st.at[j]], gk, sg[k]).wait()
            pltpu.make_async_copy(seq_dense.at[b], dk, sd[k]).wait()

            for cc in range(D // L):
                srow[pl.ds(cc * L, L)] = (
                    gsall[NNZ_SENT * j, pl.ds(cc * L, L)]
                    + gsall[NNZ_SENT * j + 1, pl.ds(cc * L, L)]
                    + gsall[NNZ_SENT * j + 2, pl.ds(cc * L, L)]
                    + gsall[NNZ_SENT * j + 3, pl.ds(cc * L, L)])
            for cc in range(DU // L):
                srow[pl.ds(D + cc * L, L)] = dsall[j, pl.ds(cc * L, L)]

            @pl.when(j >= NBUF)
            def _():
                pltpu.make_async_copy(
                    ok, comb.at[pl.ds((b - NBUF) * BLK, BLK)], so[k]).wait()

            len_v = lens_bst[j, pl.ds(0, L)]

            def t_body(t, c2):
                tv = jnp.full((L,), t, jnp.int32)
                m_lt = jnp.where(tv < len_v, one_f, zero_s)
                m_eq = jnp.where(tv == len_v, one_f, zero_s)
                for cc in range(D // L):
                    v = (gk[2 * t, pl.ds(cc * L, L)]
                         + gk[2 * t + 1, pl.ds(cc * L, L)])
                    ok[pl.ds(t * U + cc * L, L)] = (
                        v * m_lt + srow[pl.ds(cc * L, L)] * m_eq)
                for cc in range(DU // L):
                    ok[pl.ds(t * U + D + cc * L, L)] = (
                        dk[t, pl.ds(cc * L, L)] * m_lt
                        + srow[pl.ds(D + cc * L, L)] * m_eq)
                return c2

            lax.fori_loop(0, T, t_body, 0)

            for cc in range(U // L):
                ok[pl.ds(T * U + cc * L, L)] = zero_f

            lp1 = jnp.full((L,), 1, jnp.int32) + len_v
            for kk in range(4):
                tvec = kk * L + iota
                m = jnp.where(tvec < lp1, one_f, zero_s)
                mst[pl.ds(j * TP1 + kk * L, L)] = m

            pltpu.async_copy(ok, comb.at[pl.ds(b * BLK, BLK)], so[k])

            @pl.when(j + NBUF < BPW)
            def _():
                pltpu.async_copy(wseq.at[idx_st.at[j + NBUF]], gk, sg[k])
                pltpu.async_copy(seq_dense.at[b + NBUF], dk, sd[k])
        return carry

    lax.fori_loop(0, BPW // NBUF, pair_body, 0)

    for k in range(NBUF):
        pltpu.make_async_copy(
            obuf[k],
            comb.at[pl.ds((b0 + BPW - NBUF + k) * BLK, BLK)], so[k]).wait()
    pltpu.sync_copy(mst.at[pl.ds(0, BPW * TP1)],
                    masko.at[pl.ds(b0 * TP1, BPW * TP1)])


@jax.jit
def _run(seq_idx, seq_dense, sent_idx, sent_dense, lens_b, wseq, wsent):
    mesh = plsc.VectorSubcoreMesh(core_axis_name="c", subcore_axis_name="s")
    return pl.kernel(
        _sc_body,
        mesh=mesh,
        out_type=[
            jax.ShapeDtypeStruct((B * BLK,), jnp.float32),
            jax.ShapeDtypeStruct((B * TP1,), jnp.float32),
        ],
        scratch_types=[
            pltpu.VMEM((BPW, NIDX), jnp.int32),
            pltpu.VMEM((BPW * NNZ_SENT,), jnp.int32),
            pltpu.VMEM((BPW, L), jnp.int32),
            pltpu.VMEM((BPW * NNZ_SENT, D), jnp.float32),
            pltpu.VMEM((BPW, DU), jnp.float32),
            pltpu.VMEM((NIDX, D), jnp.float32),
            pltpu.VMEM((NIDX, D), jnp.float32),
            pltpu.VMEM((T, DU), jnp.float32),
            pltpu.VMEM((T, DU), jnp.float32),
            pltpu.VMEM((BLK,), jnp.float32),
            pltpu.VMEM((BLK,), jnp.float32),
            pltpu.VMEM((U,), jnp.float32),
            pltpu.VMEM((BPW * TP1 + L,), jnp.float32),
            pltpu.SemaphoreType.DMA,
            pltpu.SemaphoreType.DMA,
            pltpu.SemaphoreType.DMA,
            pltpu.SemaphoreType.DMA,
            pltpu.SemaphoreType.DMA,
            pltpu.SemaphoreType.DMA,
            pltpu.SemaphoreType.DMA,
        ],
    )(seq_idx, seq_dense, sent_idx, sent_dense, lens_b, wseq, wsent)


def kernel(seq_sparse_idx, seq_dense, sent_sparse_idx, sent_dense,
           sequence_feature_lengths, W_seq, W_sent):
    seq_idx = seq_sparse_idx.reshape(B, NIDX).astype(jnp.int32)
    sent_idx = sent_sparse_idx.reshape(B * NNZ_SENT).astype(jnp.int32)
    sent_dense2 = sent_dense.reshape(B, DU)
    lens = sequence_feature_lengths.astype(jnp.int32)
    lens_b = jnp.broadcast_to(lens[:, None], (B, L))
    comb_flat, mask_flat = _run(seq_idx, seq_dense, sent_idx, sent_dense2,
                                lens_b, W_seq, W_sent)
    return comb_flat.reshape(B, TP1, U), mask_flat.reshape(B, TP1, 1)


# speedup vs baseline: 2.1281x; 1.6643x over previous
import jax
import jax.numpy as jnp
from jax import lax
from jax.experimental import pallas as pl
from jax.experimental.pallas import tpu as pltpu
from jax.experimental.pallas import tpu_sc as plsc

B, T, V, D, DU = 1024, 50, 100000, 128, 256
U = D + DU
TP1 = T + 1
BLK = TP1 * U
NNZ_SEQ = 2
NNZ_SENT = 4
NIDX = T * NNZ_SEQ
NW = 32
BPW = B // NW
L = 16
NBUF = 2


def _sc_body(seq_idx, seq_dense, sent_idx, sent_dense, lens_b,
             wseq, wsent, comb, masko,
             idx_st, sidx_st, lens_bst, gsall, dsall,
             g0, g1, d0, d1, o0, o1, mst,
             sgs, sg0, sg1, sd0, sd1, so0, so1):
    cid = lax.axis_index("c")
    sid = lax.axis_index("s")
    wid = sid * 2 + cid
    b0 = wid * BPW

    gbuf = (g0, g1)
    dbuf = (d0, d1)
    obuf = (o0, o1)
    sg = (sg0, sg1)
    sd = (sd0, sd1)
    so = (so0, so1)

    pltpu.sync_copy(seq_idx.at[pl.ds(b0, BPW)], idx_st)
    pltpu.sync_copy(sent_idx.at[pl.ds(b0 * NNZ_SENT, BPW * NNZ_SENT)],
                    sidx_st)
    pltpu.sync_copy(lens_b.at[pl.ds(b0, BPW)], lens_bst)
    pltpu.sync_copy(sent_dense.at[pl.ds(b0, BPW)], dsall)
    pltpu.async_copy(wsent.at[sidx_st], gsall, sgs).wait()

    iota = jnp.arange(L, dtype=jnp.int32)
    zero_f = jnp.zeros((L,), jnp.float32)
    one_f = jnp.float32(1.0)
    zero_s = jnp.float32(0.0)

    for k in range(NBUF):
        pltpu.async_copy(wseq.at[idx_st.at[k]], gbuf[k], sg[k])
        pltpu.async_copy(seq_dense.at[b0 + k], dbuf[k], sd[k])

    def pair_body(p, carry):
        for k in range(NBUF):
            j = p * NBUF + k
            b = b0 + j
            gk, dk, ok = gbuf[k], dbuf[k], obuf[k]

            pltpu.make_async_copy(wseq.at[idx_st.at[j]], gk, sg[k]).wait()
            pltpu.make_async_copy(seq_dense.at[b], dk, sd[k]).wait()

            srow_v = [
                (gsall[NNZ_SENT * j, pl.ds(cc * L, L)]
                 + gsall[NNZ_SENT * j + 1, pl.ds(cc * L, L)])
                + (gsall[NNZ_SENT * j + 2, pl.ds(cc * L, L)]
                   + gsall[NNZ_SENT * j + 3, pl.ds(cc * L, L)])
                for cc in range(D // L)
            ] + [dsall[j, pl.ds(cc * L, L)] for cc in range(DU // L)]

            @pl.when(j >= NBUF)
            def _():
                pltpu.make_async_copy(
                    ok, comb.at[pl.ds((b - NBUF) * BLK, BLK)], so[k]).wait()

            len_v = lens_bst[j, pl.ds(0, L)]

            @plsc.parallel_loop(0, T, unroll=2)
            def t_body(t):
                tv = jnp.full((L,), t, jnp.int32)
                m_lt = tv < len_v
                m_eq = tv == len_v
                for cc in range(D // L):
                    v = (gk[2 * t, pl.ds(cc * L, L)]
                         + gk[2 * t + 1, pl.ds(cc * L, L)])
                    ok[pl.ds(t * U + cc * L, L)] = jnp.where(
                        m_lt, v, jnp.where(m_eq, srow_v[cc], zero_f))
                for cc in range(DU // L):
                    ok[pl.ds(t * U + D + cc * L, L)] = jnp.where(
                        m_lt, dk[t, pl.ds(cc * L, L)],
                        jnp.where(m_eq, srow_v[D // L + cc], zero_f))

            for cc in range(U // L):
                ok[pl.ds(T * U + cc * L, L)] = zero_f

            lp1 = jnp.full((L,), 1, jnp.int32) + len_v
            for kk in range(4):
                tvec = kk * L + iota
                m = jnp.where(tvec < lp1, one_f, zero_s)
                mst[pl.ds(j * TP1 + kk * L, L)] = m

            pltpu.async_copy(ok, comb.at[pl.ds(b * BLK, BLK)], so[k])

            @pl.when(j + NBUF < BPW)
            def _():
                pltpu.async_copy(wseq.at[idx_st.at[j + NBUF]], gk, sg[k])
                pltpu.async_copy(seq_dense.at[b + NBUF], dk, sd[k])
        return carry

    lax.fori_loop(0, BPW // NBUF, pair_body, 0)

    for k in range(NBUF):
        pltpu.make_async_copy(
            obuf[k],
            comb.at[pl.ds((b0 + BPW - NBUF + k) * BLK, BLK)], so[k]).wait()
    pltpu.sync_copy(mst.at[pl.ds(0, BPW * TP1)],
                    masko.at[pl.ds(b0 * TP1, BPW * TP1)])


@jax.jit
def _run(seq_idx, seq_dense, sent_idx, sent_dense, lens_b, wseq, wsent):
    mesh = plsc.VectorSubcoreMesh(core_axis_name="c", subcore_axis_name="s")
    return pl.kernel(
        _sc_body,
        mesh=mesh,
        out_type=[
            jax.ShapeDtypeStruct((B * BLK,), jnp.float32),
            jax.ShapeDtypeStruct((B * TP1,), jnp.float32),
        ],
        scratch_types=[
            pltpu.VMEM((BPW, NIDX), jnp.int32),
            pltpu.VMEM((BPW * NNZ_SENT,), jnp.int32),
            pltpu.VMEM((BPW, L), jnp.int32),
            pltpu.VMEM((BPW * NNZ_SENT, D), jnp.float32),
            pltpu.VMEM((BPW, DU), jnp.float32),
            pltpu.VMEM((NIDX, D), jnp.float32),
            pltpu.VMEM((NIDX, D), jnp.float32),
            pltpu.VMEM((T, DU), jnp.float32),
            pltpu.VMEM((T, DU), jnp.float32),
            pltpu.VMEM((BLK,), jnp.float32),
            pltpu.VMEM((BLK,), jnp.float32),
            pltpu.VMEM((BPW * TP1 + L,), jnp.float32),
            pltpu.SemaphoreType.DMA,
            pltpu.SemaphoreType.DMA,
            pltpu.SemaphoreType.DMA,
            pltpu.SemaphoreType.DMA,
            pltpu.SemaphoreType.DMA,
            pltpu.SemaphoreType.DMA,
            pltpu.SemaphoreType.DMA,
        ],
    )(seq_idx, seq_dense, sent_idx, sent_dense, lens_b, wseq, wsent)


def kernel(seq_sparse_idx, seq_dense, sent_sparse_idx, sent_dense,
           sequence_feature_lengths, W_seq, W_sent):
    seq_idx = seq_sparse_idx.reshape(B, NIDX).astype(jnp.int32)
    sent_idx = sent_sparse_idx.reshape(B * NNZ_SENT).astype(jnp.int32)
    sent_dense2 = sent_dense.reshape(B, DU)
    lens = sequence_feature_lengths.astype(jnp.int32)
    lens_b = jnp.broadcast_to(lens[:, None], (B, L))
    comb_flat, mask_flat = _run(seq_idx, seq_dense, sent_idx, sent_dense2,
                                lens_b, W_seq, W_sent)
    return comb_flat.reshape(B, TP1, U), mask_flat.reshape(B, TP1, 1)


# speedup vs baseline: 2.8563x; 1.3422x over previous
import jax
import jax.numpy as jnp
from jax import lax
from jax.experimental import pallas as pl
from jax.experimental.pallas import tpu as pltpu
from jax.experimental.pallas import tpu_sc as plsc

B, T, V, D, DU = 1024, 50, 100000, 128, 256
U = D + DU
TP1 = T + 1
BLK = TP1 * U
NNZ_SEQ = 2
NNZ_SENT = 4
NIDX = T * NNZ_SEQ
NW = 32
BPW = B // NW
L = 16
NBUF = 2


def _sc_body(seq_idx, seq_dense, sent_idx, sent_dense, lens_b,
             wseq, wsent, comb, masko,
             idx_st, sidx_st, lens_bst, gsall, dsall,
             g0, g1, d0, d1, o0, o1, mst,
             sgs, sg0, sg1, sd0, sd1, so0, so1):
    cid = lax.axis_index("c")
    sid = lax.axis_index("s")
    wid = sid * 2 + cid
    b0 = wid * BPW

    gbuf = (g0, g1)
    dbuf = (d0, d1)
    obuf = (o0, o1)
    sg = (sg0, sg1)
    sd = (sd0, sd1)
    so = (so0, so1)

    pltpu.sync_copy(seq_idx.at[pl.ds(b0, BPW)], idx_st)
    pltpu.sync_copy(sent_idx.at[pl.ds(b0 * NNZ_SENT, BPW * NNZ_SENT)],
                    sidx_st)
    pltpu.sync_copy(lens_b.at[pl.ds(b0, BPW)], lens_bst)
    pltpu.sync_copy(sent_dense.at[pl.ds(b0, BPW)], dsall)

    iota = jnp.arange(L, dtype=jnp.int32)
    zero_f = jnp.zeros((L,), jnp.float32)
    one_f = jnp.float32(1.0)
    zero_s = jnp.float32(0.0)

    for k in range(NBUF):
        pltpu.async_copy(wseq.at[idx_st.at[k]], gbuf[k], sg[k])
        pltpu.async_copy(seq_dense.at[b0 + k], dbuf[k], sd[k])

    def make_pair_body(gi):
        def pair_body(p, carry):
            for k in range(NBUF):
                j = p * NBUF + k
                b = b0 + j
                jl = j - gi * (BPW // 2)
                gk, dk, ok = gbuf[k], dbuf[k], obuf[k]

                pltpu.make_async_copy(wseq.at[idx_st.at[j]], gk, sg[k]).wait()
                pltpu.make_async_copy(seq_dense.at[b], dk, sd[k]).wait()

                srow_v = [
                    (gsall[NNZ_SENT * jl, pl.ds(cc * L, L)]
                     + gsall[NNZ_SENT * jl + 1, pl.ds(cc * L, L)])
                    + (gsall[NNZ_SENT * jl + 2, pl.ds(cc * L, L)]
                       + gsall[NNZ_SENT * jl + 3, pl.ds(cc * L, L)])
                    for cc in range(D // L)
                ] + [dsall[j, pl.ds(cc * L, L)] for cc in range(DU // L)]

                @pl.when(j >= NBUF)
                def _():
                    pltpu.make_async_copy(ok, comb.at[b - NBUF], so[k]).wait()

                len_v = lens_bst[j, pl.ds(0, L)]

                @plsc.parallel_loop(0, T, unroll=2)
                def t_body(t):
                    tv = jnp.full((L,), t, jnp.int32)
                    m_lt = tv < len_v
                    m_eq = tv == len_v
                    for cc in range(D // L):
                        v = (gk[2 * t, pl.ds(cc * L, L)]
                             + gk[2 * t + 1, pl.ds(cc * L, L)])
                        ok[t, pl.ds(cc * L, L)] = jnp.where(
                            m_lt, v, jnp.where(m_eq, srow_v[cc], zero_f))
                    for cc in range(DU // L):
                        ok[t, pl.ds(D + cc * L, L)] = jnp.where(
                            m_lt, dk[t, pl.ds(cc * L, L)],
                            jnp.where(m_eq, srow_v[D // L + cc], zero_f))

                for cc in range(U // L):
                    ok[T, pl.ds(cc * L, L)] = zero_f

                lp1 = jnp.full((L,), 1, jnp.int32) + len_v
                for kk in range(4):
                    tvec = kk * L + iota
                    m = jnp.where(tvec < lp1, one_f, zero_s)
                    mst[pl.ds(j * TP1 + kk * L, L)] = m

                pltpu.async_copy(ok, comb.at[b], so[k])

                @pl.when(j + NBUF < BPW)
                def _():
                    pltpu.async_copy(wseq.at[idx_st.at[j + NBUF]], gk, sg[k])
                    pltpu.async_copy(seq_dense.at[b + NBUF], dk, sd[k])
            return carry
        return pair_body

    half_pairs = BPW // 2 // NBUF
    for gi in range(2):
        pltpu.async_copy(
            wsent.at[sidx_st.at[pl.ds(gi * (BPW // 2) * NNZ_SENT,
                                      (BPW // 2) * NNZ_SENT)]],
            gsall, sgs).wait()
        lax.fori_loop(gi * half_pairs, (gi + 1) * half_pairs,
                      make_pair_body(gi), 0)

    for k in range(NBUF):
        pltpu.make_async_copy(
            obuf[k], comb.at[b0 + BPW - NBUF + k], so[k]).wait()
    pltpu.sync_copy(mst.at[pl.ds(0, BPW * TP1)],
                    masko.at[pl.ds(b0 * TP1, BPW * TP1)])


@jax.jit
def _run(seq_idx, seq_dense, sent_idx, sent_dense, lens_b, wseq, wsent):
    mesh = plsc.VectorSubcoreMesh(core_axis_name="c", subcore_axis_name="s")
    return pl.kernel(
        _sc_body,
        mesh=mesh,
        out_type=[
            jax.ShapeDtypeStruct((B, TP1, U), jnp.float32),
            jax.ShapeDtypeStruct((B * TP1,), jnp.float32),
        ],
        scratch_types=[
            pltpu.VMEM((BPW, NIDX), jnp.int32),
            pltpu.VMEM((BPW * NNZ_SENT,), jnp.int32),
            pltpu.VMEM((BPW, L), jnp.int32),
            pltpu.VMEM((BPW * NNZ_SENT // 2, D), jnp.float32),
            pltpu.VMEM((BPW, DU), jnp.float32),
            pltpu.VMEM((NIDX, D), jnp.float32),
            pltpu.VMEM((NIDX, D), jnp.float32),
            pltpu.VMEM((T, DU), jnp.float32),
            pltpu.VMEM((T, DU), jnp.float32),
            pltpu.VMEM((TP1, U), jnp.float32),
            pltpu.VMEM((TP1, U), jnp.float32),
            pltpu.VMEM((BPW * TP1 + L,), jnp.float32),
            pltpu.SemaphoreType.DMA,
            pltpu.SemaphoreType.DMA,
            pltpu.SemaphoreType.DMA,
            pltpu.SemaphoreType.DMA,
            pltpu.SemaphoreType.DMA,
            pltpu.SemaphoreType.DMA,
            pltpu.SemaphoreType.DMA,
        ],
    )(seq_idx, seq_dense, sent_idx, sent_dense, lens_b, wseq, wsent)


def kernel(seq_sparse_idx, seq_dense, sent_sparse_idx, sent_dense,
           sequence_feature_lengths, W_seq, W_sent):
    seq_idx = seq_sparse_idx.reshape(B, NIDX).astype(jnp.int32)
    sent_idx = sent_sparse_idx.reshape(B * NNZ_SENT).astype(jnp.int32)
    sent_dense2 = sent_dense.reshape(B, DU)
    lens = sequence_feature_lengths.astype(jnp.int32)
    lens_b = jnp.broadcast_to(lens[:, None], (B, L))
    comb, mask_flat = _run(seq_idx, seq_dense, sent_idx, sent_dense2,
                           lens_b, W_seq, W_sent)
    return comb, mask_flat.reshape(B, TP1, 1)
